# baseline (speedup 1.0000x reference)
import functools

import jax
import jax.numpy as jnp
from jax import lax
from jax.experimental import pallas as pl
from jax.experimental.pallas import tpu as pltpu
from jax.experimental.pallas import tpu_sc as plsc

_V = 100000
_D = 32
_B = 4096
_L = 50
_F = 4
_N = _B * _L

_NC = 2
_NS = 16
_NW = _NC * _NS
_PER_W = _N // _NW
_C = 128
_NCH = _PER_W // _C


def _sc_body(i0, i1, i2, i3, e0, e1, e2, e3, out, idx_v, rows_v, sem):
    wid = lax.axis_index("s") * _NC + lax.axis_index("c")
    base = wid * _PER_W

    for f, ih in enumerate((i0, i1, i2, i3)):
        pltpu.sync_copy(ih.at[pl.ds(wid * _NCH, _NCH)], idx_v.at[f])

    def chunk(ci, _):
        copies = []
        for f, eh in enumerate((e0, e1, e2, e3)):
            copies.append(
                pltpu.async_copy(eh.at[idx_v.at[f, ci]], rows_v.at[f], sem))
        for c in copies:
            c.wait()
        off = base + ci * _C
        for f in range(_F):
            pltpu.sync_copy(rows_v.at[f],
                            out.at[pl.ds(off, _C), pl.ds(f * _D, _D)])
        return 0

    lax.fori_loop(0, _NCH, chunk, 0)


@functools.partial(
    pl.kernel,
    out_type=jax.ShapeDtypeStruct((_N, _F * _D), jnp.float32),
    mesh=plsc.VectorSubcoreMesh(core_axis_name="c", subcore_axis_name="s"),
    compiler_params=pltpu.CompilerParams(use_tc_tiling_on_sc=False),
    scratch_types=[
        pltpu.VMEM((_F, _NCH, _C), jnp.int32),
        pltpu.VMEM((_F, _C, _D), jnp.float32),
        pltpu.SemaphoreType.DMA,
    ],
)
def _multi_gather(i0, i1, i2, i3, e0, e1, e2, e3, out, idx_v, rows_v, sem):
    _sc_body(i0, i1, i2, i3, e0, e1, e2, e3, out, idx_v, rows_v, sem)


def kernel(idx_cat0, idx_cat1, idx_cat2, idx_cat3,
           emb_cat0, emb_cat1, emb_cat2, emb_cat3):
    idxs = [i.reshape(_NW * _NCH, _C).astype(jnp.int32)
            for i in (idx_cat0, idx_cat1, idx_cat2, idx_cat3)]
    out = _multi_gather(*idxs, emb_cat0, emb_cat1, emb_cat2, emb_cat3)
    return out.reshape(_B, _L, _F * _D)


# speedup vs baseline: 1.6625x; 1.6625x over previous
import functools

import jax
import jax.numpy as jnp
from jax import lax
from jax.experimental import pallas as pl
from jax.experimental.pallas import tpu as pltpu
from jax.experimental.pallas import tpu_sc as plsc

_V = 100000
_D = 32
_B = 4096
_L = 50
_F = 4
_N = _B * _L

_NC = 2
_NS = 16
_NW = _NC * _NS
_PER_W = _N // _NW
_C = 128
_NCH = _PER_W // _C


def _sc_body(i0, i1, i2, i3, e0, e1, e2, e3, out, idx_v, rows_v, sem):
    wid = lax.axis_index("s") * _NC + lax.axis_index("c")
    base = wid * _PER_W

    for f, ih in enumerate((i0, i1, i2, i3)):
        pltpu.sync_copy(ih.at[pl.ds(wid * _NCH, _NCH)], idx_v.at[f])

    def chunk(ci, _):
        copies = []
        for f, eh in enumerate((e0, e1, e2, e3)):
            copies.append(
                pltpu.async_copy(eh.at[idx_v.at[f, ci]], rows_v.at[f], sem))
        for c in copies:
            c.wait()
        off = base + ci * _C
        for f in range(_F):
            pltpu.sync_copy(rows_v.at[f],
                            out.at[pl.ds(off, _C), pl.ds(f * _D, _D)])
        return 0

    lax.fori_loop(0, _NCH, chunk, 0)


@functools.partial(
    pl.kernel,
    out_type=jax.ShapeDtypeStruct((_N, _F * _D), jnp.float32),
    mesh=plsc.VectorSubcoreMesh(core_axis_name="c", subcore_axis_name="s"),
    compiler_params=pltpu.CompilerParams(use_tc_tiling_on_sc=False),
    scratch_types=[
        pltpu.VMEM((_F, _NCH, _C), jnp.int32),
        pltpu.VMEM((_F, _C, _D), jnp.float32),
        pltpu.SemaphoreType.DMA,
    ],
)
def _multi_gather(i0, i1, i2, i3, e0, e1, e2, e3, out, idx_v, rows_v, sem):
    _sc_body(i0, i1, i2, i3, e0, e1, e2, e3, out, idx_v, rows_v, sem)


def kernel(idx_cat0, idx_cat1, idx_cat2, idx_cat3,
           emb_cat0, emb_cat1, emb_cat2, emb_cat3):
    idxs = [i.T.reshape(_NW * _NCH, _C).astype(jnp.int32)
            for i in (idx_cat0, idx_cat1, idx_cat2, idx_cat3)]
    out = _multi_gather(*idxs, emb_cat0, emb_cat1, emb_cat2, emb_cat3)
    return out.reshape(_L, _B, _F * _D).transpose(1, 0, 2)


# speedup vs baseline: 1.8493x; 1.1124x over previous
import functools

import jax
import jax.numpy as jnp
from jax import lax
from jax.experimental import pallas as pl
from jax.experimental.pallas import tpu as pltpu
from jax.experimental.pallas import tpu_sc as plsc

_V = 100000
_D = 32
_B = 4096
_L = 50
_F = 4
_N = _B * _L

_NC = 2
_NS = 16
_NW = _NC * _NS
_PER_W = _N // _NW
_C = 128
_NCH = _PER_W // _C


def _sc_body(i0, i1, i2, i3, e0, e1, e2, e3, out, idx_v,
             rows_a, rows_b, gsem_a, gsem_b, wsem_a, wsem_b):
    embs = (e0, e1, e2, e3)
    wid = lax.axis_index("s") * _NC + lax.axis_index("c")
    base = wid * _PER_W

    for f, ih in enumerate((i0, i1, i2, i3)):
        pltpu.sync_copy(ih.at[pl.ds(wid * _NCH, _NCH)], idx_v.at[f])

    def fire_gather(ci, rows, gsem):
        for f in range(_F):
            pltpu.async_copy(embs[f].at[idx_v.at[f, ci]], rows.at[f], gsem)

    def wait_gather(rows, gsem):
        for f in range(_F):
            pltpu.make_async_copy(embs[f].at[pl.ds(0, _C)], rows.at[f],
                                  gsem).wait()

    def fire_write(ci, rows, wsem):
        off = base + ci * _C
        for f in range(_F):
            pltpu.async_copy(rows.at[f],
                             out.at[pl.ds(off, _C), pl.ds(f * _D, _D)], wsem)

    def wait_write(rows, wsem):
        for f in range(_F):
            pltpu.make_async_copy(rows.at[f],
                                  out.at[pl.ds(0, _C), pl.ds(f * _D, _D)],
                                  wsem).wait()

    fire_gather(0, rows_a, gsem_a)
    fire_gather(1, rows_b, gsem_b)

    def body(j, _):
        ca = 2 * j
        wait_gather(rows_a, gsem_a)
        fire_write(ca, rows_a, wsem_a)
        wait_gather(rows_b, gsem_b)
        fire_write(ca + 1, rows_b, wsem_b)
        wait_write(rows_a, wsem_a)

        @pl.when(ca + 2 < _NCH)
        def _():
            fire_gather(ca + 2, rows_a, gsem_a)

        wait_write(rows_b, wsem_b)

        @pl.when(ca + 3 < _NCH)
        def _():
            fire_gather(ca + 3, rows_b, gsem_b)

        return 0

    lax.fori_loop(0, _NCH // 2, body, 0)


@functools.partial(
    pl.kernel,
    out_type=jax.ShapeDtypeStruct((_N, _F * _D), jnp.float32),
    mesh=plsc.VectorSubcoreMesh(core_axis_name="c", subcore_axis_name="s"),
    compiler_params=pltpu.CompilerParams(use_tc_tiling_on_sc=False),
    scratch_types=[
        pltpu.VMEM((_F, _NCH, _C), jnp.int32),
        pltpu.VMEM((_F, _C, _D), jnp.float32),
        pltpu.VMEM((_F, _C, _D), jnp.float32),
        pltpu.SemaphoreType.DMA,
        pltpu.SemaphoreType.DMA,
        pltpu.SemaphoreType.DMA,
        pltpu.SemaphoreType.DMA,
    ],
)
def _multi_gather(i0, i1, i2, i3, e0, e1, e2, e3, out, idx_v,
                  rows_a, rows_b, gsem_a, gsem_b, wsem_a, wsem_b):
    _sc_body(i0, i1, i2, i3, e0, e1, e2, e3, out, idx_v,
             rows_a, rows_b, gsem_a, gsem_b, wsem_a, wsem_b)


def kernel(idx_cat0, idx_cat1, idx_cat2, idx_cat3,
           emb_cat0, emb_cat1, emb_cat2, emb_cat3):
    idxs = [i.T.reshape(_NW * _NCH, _C).astype(jnp.int32)
            for i in (idx_cat0, idx_cat1, idx_cat2, idx_cat3)]
    out = _multi_gather(*idxs, emb_cat0, emb_cat1, emb_cat2, emb_cat3)
    return out.reshape(_L, _B, _F * _D).transpose(1, 0, 2)
